# baseline (speedup 1.0000x reference)
import functools
import math

import jax
import jax.numpy as jnp
from jax import lax
from jax.experimental import pallas as pl
from jax.experimental.pallas import tpu as pltpu
from jax.experimental.pallas import tpu_sc as plsc

D_MODEL = 64
SCALE = math.sqrt(D_MODEL)

IDX_W = 128
CHUNK = 512
GATHERS_PER_CHUNK = CHUNK // IDX_W


def _make_sc_embed(n_idx: int, vocab: int):
    info = plsc.get_sparse_core_info()
    nc, ns, nl = info.num_cores, info.num_subcores, info.num_lanes
    nw = nc * ns
    assert n_idx % (nw * CHUNK) == 0
    per_w = n_idx // nw
    n_chunks = per_w // CHUNK
    idx_rows_w = per_w // IDX_W

    mesh = plsc.VectorSubcoreMesh(core_axis_name="c", subcore_axis_name="s")

    @functools.partial(
        pl.kernel,
        out_type=jax.ShapeDtypeStruct((n_idx, D_MODEL), jnp.float32),
        mesh=mesh,
        scratch_types=[
            pltpu.VMEM((idx_rows_w, IDX_W), jnp.int32),
            pltpu.VMEM((CHUNK, D_MODEL), jnp.float32),
            pltpu.SemaphoreType.DMA,
        ],
        compiler_params=pltpu.CompilerParams(use_tc_tiling_on_sc=False),
    )
    def sc_embed(idx_hbm, table_hbm, out_hbm, idx_v, buf, sem):
        wid = lax.axis_index("s") * nc + lax.axis_index("c")
        pltpu.sync_copy(idx_hbm.at[pl.ds(wid * idx_rows_w, idx_rows_w)], idx_v)
        row_base = wid * per_w

        def chunk_body(t, carry):
            handles = []
            for k in range(GATHERS_PER_CHUNK):
                h = pltpu.async_copy(
                    table_hbm.at[idx_v.at[t * GATHERS_PER_CHUNK + k]],
                    buf.at[pl.ds(k * IDX_W, IDX_W)],
                    sem,
                )
                handles.append(h)
            for h in handles:
                h.wait()

            def scale_row(r, c2):
                for c in range(D_MODEL // nl):
                    sl = pl.ds(c * nl, nl)
                    buf[r, sl] = buf[r, sl] * SCALE
                return c2

            lax.fori_loop(0, CHUNK, scale_row, 0)

            pltpu.sync_copy(buf, out_hbm.at[pl.ds(row_base + t * CHUNK, CHUNK)])
            return carry

        lax.fori_loop(0, n_chunks, chunk_body, 0)

    return sc_embed


def kernel(x, embed_weight):
    b, s = x.shape
    vocab, d = embed_weight.shape
    n_idx = b * s
    idx = x.reshape(n_idx // IDX_W, IDX_W).astype(jnp.int32)
    out = _make_sc_embed(n_idx, vocab)(idx, embed_weight)
    return out.reshape(b, s, d)
